# baseline (speedup 1.0000x reference)
import functools

import jax
import jax.numpy as jnp
from jax import lax
from jax.experimental import pallas as pl
from jax.experimental.pallas import tpu as pltpu
from jax.experimental.pallas import tpu_sc as plsc

N = 10000
E = 320000
D = 128
H = 128

NC = 2
NS = 16
NW = NC * NS
EPW = E // NW
CH = 80
NCHUNK = EPW // CH
CHP = 80
NCHUNKP = EPW // CHP
NGRP = CHP // 16
RPS = 624
RTAIL = N - NS * RPS

_MESH = plsc.VectorSubcoreMesh(
    core_axis_name="c", subcore_axis_name="s", num_cores=NC, num_subcores=NS)


def _agg_pipeline(h_hbm, src_hbm, dst_hbm, agg_sh, base,
                  sidx, didx, rows, gsem, ssem, on_chunk, drain_chunk):
  def start_g(k, b):
    off = pl.multiple_of(base + k * CH, 8)
    pltpu.sync_copy(src_hbm.at[pl.ds(off, CH)], sidx[b])
    pltpu.sync_copy(dst_hbm.at[pl.ds(off, CH)], didx[b])
    pltpu.async_copy(h_hbm.at[sidx[b]], rows[b], gsem[b])

  def scat(k, b):
    pltpu.make_async_copy(h_hbm.at[sidx[b]], rows[b], gsem[b]).wait()
    pltpu.async_copy(rows[b], agg_sh.at[didx[b]], ssem[b], add=True)
    on_chunk(b)

  def drain(b):
    pltpu.make_async_copy(rows[b], agg_sh.at[didx[b]], ssem[b]).wait()
    drain_chunk(b)

  start_g(0, 0)
  start_g(1, 1)
  nstep = (NCHUNK - 1) // 4

  def step(j, carry):
    for i in range(4):
      k = 4 * j + i
      b2 = (i + 2) % 4
      if i in (0, 1):
        @pl.when(j > 0)
        def _():
          drain(b2)
        start_g(k + 2, b2)
      elif i == 2:
        drain(b2)
        start_g(k + 2, b2)
      else:
        @pl.when(j < nstep - 1)
        def _():
          drain(b2)
          start_g(k + 2, b2)
      scat(k, i)
    return carry

  lax.fori_loop(0, nstep, step, 0)
  scat(NCHUNK - 1, (NCHUNK - 1) % 4)
  for b in (1, 2, 3, 0):
    drain(b)


def _agg_deg_body(h_hbm, src_hbm, dst_hbm, znd_hbm, zn_hbm, ones_hbm,
                  agg_out, deg_out,
                  sidx0, sidx1, sidx2, sidx3, didx0, didx1, didx2, didx3,
                  ones_v, rows0, rows1, rows2, rows3,
                  agg_sh, deg_sh,
                  gsem0, gsem1, gsem2, gsem3,
                  ssem0, ssem1, ssem2, ssem3,
                  dsem0, dsem1, dsem2, dsem3):
  cid = lax.axis_index("c")
  sid = lax.axis_index("s")
  wid = cid * NS + sid
  rs = pl.multiple_of(sid * RPS, 8)
  pltpu.sync_copy(znd_hbm.at[pl.ds(rs, RPS)], agg_sh.at[pl.ds(rs, RPS)])
  pltpu.sync_copy(ones_hbm, ones_v)

  @pl.when(sid == 0)
  def _():
    pltpu.sync_copy(zn_hbm, deg_sh)
    tl = pl.multiple_of(NS * RPS, 8)
    pltpu.sync_copy(znd_hbm.at[pl.ds(tl, RTAIL)], agg_sh.at[pl.ds(tl, RTAIL)])

  plsc.subcore_barrier()

  didx = (didx0, didx1, didx2, didx3)
  dsem = (dsem0, dsem1, dsem2, dsem3)

  def on_chunk(b):
    pltpu.async_copy(ones_v, deg_sh.at[didx[b]], dsem[b], add=True)

  def drain_chunk(b):
    pltpu.make_async_copy(ones_v, deg_sh.at[didx[b]], dsem[b]).wait()

  _agg_pipeline(h_hbm, src_hbm, dst_hbm, agg_sh, wid * EPW,
                (sidx0, sidx1, sidx2, sidx3), didx,
                (rows0, rows1, rows2, rows3),
                (gsem0, gsem1, gsem2, gsem3),
                (ssem0, ssem1, ssem2, ssem3),
                on_chunk, drain_chunk)
  plsc.subcore_barrier()
  pltpu.sync_copy(agg_sh.at[pl.ds(rs, RPS)], agg_out.at[cid, pl.ds(rs, RPS)])

  @pl.when(sid == 0)
  def _():
    pltpu.sync_copy(deg_sh, deg_out.at[cid])
    tl = pl.multiple_of(NS * RPS, 8)
    pltpu.sync_copy(agg_sh.at[pl.ds(tl, RTAIL)],
                    agg_out.at[cid, pl.ds(tl, RTAIL)])


def _agg_body(h_hbm, src_hbm, dst_hbm, znd_hbm,
              agg_out,
              sidx0, sidx1, sidx2, sidx3, didx0, didx1, didx2, didx3,
              rows0, rows1, rows2, rows3,
              agg_sh,
              gsem0, gsem1, gsem2, gsem3,
              ssem0, ssem1, ssem2, ssem3):
  cid = lax.axis_index("c")
  sid = lax.axis_index("s")
  wid = cid * NS + sid
  rs = pl.multiple_of(sid * RPS, 8)
  pltpu.sync_copy(znd_hbm.at[pl.ds(rs, RPS)], agg_sh.at[pl.ds(rs, RPS)])

  @pl.when(sid == 0)
  def _():
    tl = pl.multiple_of(NS * RPS, 8)
    pltpu.sync_copy(znd_hbm.at[pl.ds(tl, RTAIL)], agg_sh.at[pl.ds(tl, RTAIL)])

  plsc.subcore_barrier()
  _agg_pipeline(h_hbm, src_hbm, dst_hbm, agg_sh, wid * EPW,
                (sidx0, sidx1, sidx2, sidx3), (didx0, didx1, didx2, didx3),
                (rows0, rows1, rows2, rows3),
                (gsem0, gsem1, gsem2, gsem3),
                (ssem0, ssem1, ssem2, ssem3),
                lambda b: None, lambda b: None)
  plsc.subcore_barrier()
  pltpu.sync_copy(agg_sh.at[pl.ds(rs, RPS)], agg_out.at[cid, pl.ds(rs, RPS)])

  @pl.when(sid == 0)
  def _():
    tl = pl.multiple_of(NS * RPS, 8)
    pltpu.sync_copy(agg_sh.at[pl.ds(tl, RTAIL)],
                    agg_out.at[cid, pl.ds(tl, RTAIL)])


_agg_deg_call = pl.kernel(
    _agg_deg_body,
    out_type=[
        jax.ShapeDtypeStruct((NC, N, D), jnp.float32),
        jax.ShapeDtypeStruct((NC, N), jnp.float32),
    ],
    mesh=_MESH,
    scratch_types=(
        [pltpu.VMEM((CH,), jnp.int32) for _ in range(8)]
        + [pltpu.VMEM((CH,), jnp.float32)]
        + [pltpu.VMEM((CH, D), jnp.float32) for _ in range(4)]
        + [
            pltpu.VMEM_SHARED((N, D), jnp.float32),
            pltpu.VMEM_SHARED((N,), jnp.float32),
        ]
        + [pltpu.SemaphoreType.DMA for _ in range(12)]
    ),
)

_agg_call = pl.kernel(
    _agg_body,
    out_type=jax.ShapeDtypeStruct((NC, N, D), jnp.float32),
    mesh=_MESH,
    scratch_types=(
        [pltpu.VMEM((CH,), jnp.int32) for _ in range(8)]
        + [pltpu.VMEM((CH, D), jnp.float32) for _ in range(4)]
        + [pltpu.VMEM_SHARED((N, D), jnp.float32)]
        + [pltpu.SemaphoreType.DMA for _ in range(8)]
    ),
)


def _pair_body(a_hbm, b_hbm, ps_hbm, pd_hbm, ef_hbm, wrow_hbm, wm2_hbm,
               out_hbm,
               pidx0, pidx1, pidx2, pidx3,
               qidx0, qidx1, qidx2, qidx3,
               rows0, rows1, rows2, rows3,
               efv0, efv1, wv, mv, tv0, tv1,
               gsem0, gsem1, gsem2, gsem3,
               bsem0, bsem1, bsem2, bsem3,
               esem0, esem1, wsem0, wsem1):
  cid = lax.axis_index("c")
  sid = lax.axis_index("s")
  base = (cid * NS + sid) * EPW
  pidx = (pidx0, pidx1, pidx2, pidx3)
  qidx = (qidx0, qidx1, qidx2, qidx3)
  rows = (rows0, rows1, rows2, rows3)
  gsem = (gsem0, gsem1, gsem2, gsem3)
  bsem = (bsem0, bsem1, bsem2, bsem3)
  efv = (efv0, efv1)
  tv = (tv0, tv1)
  esem = (esem0, esem1)
  wsem = (wsem0, wsem1)

  pltpu.sync_copy(wrow_hbm, wv)
  pltpu.sync_copy(wm2_hbm, mv)
  wrow_vals = [wv[pl.ds(16 * i, 16)] for i in range(8)]
  wm2_vals = [mv[pl.ds(16 * i, 16)] for i in range(8)]

  def start_a(k, b):
    off = pl.multiple_of(base + k * CHP, 8)
    pltpu.sync_copy(ps_hbm.at[pl.ds(off, CHP)], pidx[b])
    pltpu.async_copy(a_hbm.at[pidx[b]], rows[b], gsem[b])

  def start_b(k, b, tb):
    off = pl.multiple_of(base + k * CHP, 8)
    pltpu.sync_copy(pd_hbm.at[pl.ds(off, CHP)], qidx[b])
    pltpu.make_async_copy(a_hbm.at[pidx[b]], rows[b], gsem[b]).wait()
    pltpu.async_copy(b_hbm.at[qidx[b]], rows[b], bsem[b], add=True)
    pltpu.async_copy(ef_hbm.at[pl.ds(off, CHP)], efv[tb].at[pl.ds(0, CHP)],
                     esem[tb])

  def compute(k, b, tb, wb_drain):
    off = pl.multiple_of(base + k * CHP, 8)
    pltpu.make_async_copy(b_hbm.at[qidx[b]], rows[b], bsem[b]).wait()
    pltpu.make_async_copy(ef_hbm.at[pl.ds(off, CHP)],
                          efv[tb].at[pl.ds(0, CHP)], esem[tb]).wait()

    def wdrain():
      pltpu.make_async_copy(tv[tb], out_hbm.at[pl.ds(off * 16, CHP * 16)],
                            wsem[tb]).wait()

    if wb_drain is True:
      wdrain()
    elif wb_drain is not False:
      pl.when(wb_drain)(wdrain)

    @plsc.parallel_loop(0, NGRP, unroll=1)
    def group(g):
      ef16 = efv[tb][pl.ds(pl.multiple_of(g * 16, 8), 16)]
      for le in range(16):
        e = g * 16 + le
        ef_spl = jnp.full((16,), ef16[le], jnp.float32)
        acc = None
        for j8 in range(8):
          s = rows[b][e, pl.ds(16 * j8, 16)]
          t = jnp.maximum(s + ef_spl * wrow_vals[j8], 0.0)
          p = t * wm2_vals[j8]
          acc = p if acc is None else acc + p
        tv[tb][pl.ds(pl.multiple_of(e * 16, 8), 16)] = acc
    pltpu.async_copy(tv[tb], out_hbm.at[pl.ds(off * 16, CHP * 16)], wsem[tb])

  start_a(0, 0)
  start_a(1, 1)

  def step(j, carry):
    for i in range(4):
      k = 4 * j + i
      if i == 3:
        @pl.when(j != (NCHUNKP - 1) // 4 - 1)
        def _():
          start_a(k + 2, (i + 2) % 4)
      else:
        start_a(k + 2, (i + 2) % 4)
      start_b(k, i, i % 2)
      if i == 0:
        @pl.when(j > 0)
        def _():
          compute(k - 1, 3, 1, True)
      elif i == 3:
        compute(k - 1, 2, 0, True)
      else:
        compute(k - 1, i - 1, (i - 1) % 2, j > 0)
    return carry

  lax.fori_loop(0, (NCHUNKP - 1) // 4, step, 0)
  k = NCHUNKP - 1
  off_last = pl.multiple_of(base + k * CHP, 8)
  start_b(k, k % 4, k % 2)
  compute(k - 1, (k - 1) % 4, (k - 1) % 2, True)
  compute(k, k % 4, k % 2, True)
  for tb in (0, 1):
    pltpu.make_async_copy(tv[tb], out_hbm.at[pl.ds(off_last * 16, CHP * 16)],
                          wsem[tb]).wait()


_pair_call = pl.kernel(
    _pair_body,
    out_type=jax.ShapeDtypeStruct((E * 16,), jnp.float32),
    mesh=_MESH,
    scratch_types=(
        [pltpu.VMEM((CHP,), jnp.int32) for _ in range(8)]
        + [pltpu.VMEM((CHP, D), jnp.float32) for _ in range(4)]
        + [
            pltpu.VMEM((128,), jnp.float32),
            pltpu.VMEM((128,), jnp.float32),
            pltpu.VMEM((D,), jnp.float32),
            pltpu.VMEM((D,), jnp.float32),
            pltpu.VMEM((CHP * 16,), jnp.float32),
            pltpu.VMEM((CHP * 16,), jnp.float32),
        ]
        + [pltpu.SemaphoreType.DMA for _ in range(12)]
    ),
)

BR = 2000


def _tred_body(t_ref, s_ref, bm2_ref, out_ref):
  out_ref[...] = (
      jnp.dot(t_ref[...], s_ref[...], preferred_element_type=jnp.float32)
      + bm2_ref[...])


def _tc_reduce(t2, sel, bm2):
  nrow = E * 16 // 128
  return pl.pallas_call(
      _tred_body,
      grid=(nrow // BR,),
      in_specs=[
          pl.BlockSpec((BR, 128), lambda k: (k, 0)),
          pl.BlockSpec((128, 8), lambda k: (0, 0)),
          pl.BlockSpec((1, 1), lambda k: (0, 0)),
      ],
      out_specs=pl.BlockSpec((BR, 8), lambda k: (k, 0)),
      out_shape=jax.ShapeDtypeStruct((nrow, 8), jnp.float32),
  )(t2, sel, bm2)


def _layer_body(h_ref, aggp_ref, degp_ref, ws_ref, wn_ref, b_ref, g_ref,
                be_ref, out_ref):
  agg = aggp_ref[0] + aggp_ref[1]
  deg = degp_ref[0] + degp_ref[1]
  mean = agg / jnp.maximum(deg, 1.0)[:, None]
  z = (jnp.dot(h_ref[...], ws_ref[...], preferred_element_type=jnp.float32)
       + jnp.dot(mean, wn_ref[...], preferred_element_type=jnp.float32)
       + b_ref[...])
  mu = jnp.mean(z, axis=0)
  var = jnp.mean(jnp.square(z - mu), axis=0)
  zn = (z - mu) / jnp.sqrt(var + 1e-5) * g_ref[...] + be_ref[...]
  out_ref[...] = jnp.maximum(zn, 0.0)


def _tc_layer(h, aggp, degp, ws, wn, b, g, be):
  return pl.pallas_call(
      _layer_body,
      out_shape=jax.ShapeDtypeStruct((N, H), jnp.float32),
  )(h, aggp, degp, ws, wn, b, g, be)


def _layer_ab_body(h_ref, aggp_ref, degp_ref, ws_ref, wn_ref, b_ref, g_ref,
                   be_ref, wma_ref, wmb_ref, bm1_ref, a_ref, b_out_ref):
  agg = aggp_ref[0] + aggp_ref[1]
  deg = degp_ref[0] + degp_ref[1]
  mean = agg / jnp.maximum(deg, 1.0)[:, None]
  z = (jnp.dot(h_ref[...], ws_ref[...], preferred_element_type=jnp.float32)
       + jnp.dot(mean, wn_ref[...], preferred_element_type=jnp.float32)
       + b_ref[...])
  mu = jnp.mean(z, axis=0)
  var = jnp.mean(jnp.square(z - mu), axis=0)
  zn = (z - mu) / jnp.sqrt(var + 1e-5) * g_ref[...] + be_ref[...]
  h3 = jnp.maximum(zn, 0.0)
  a_ref[...] = jnp.dot(h3, wma_ref[...], preferred_element_type=jnp.float32)
  b_out_ref[...] = (
      jnp.dot(h3, wmb_ref[...], preferred_element_type=jnp.float32)
      + bm1_ref[...])


def _tc_layer_ab(h, aggp, degp, ws, wn, b, g, be, wma, wmb, bm1):
  return pl.pallas_call(
      _layer_ab_body,
      out_shape=[
          jax.ShapeDtypeStruct((N, H), jnp.float32),
          jax.ShapeDtypeStruct((N, H), jnp.float32),
      ],
  )(h, aggp, degp, ws, wn, b, g, be, wma, wmb, bm1)


@jax.jit
def kernel(node_feats, edge_index, edge_feats, pair_src, pair_dst,
           Wself0, Wneigh0, bs0, gamma0, beta0,
           Wself1, Wneigh1, bs1, gamma1, beta1,
           Wself2, Wneigh2, bs2, gamma2, beta2,
           Wm1, bm1, Wm2, bm2):
  src = edge_index[0]
  dst = edge_index[1]
  znd = jnp.zeros((N, D), jnp.float32)
  zn = jnp.zeros((N,), jnp.float32)
  ones = jnp.ones((CH,), jnp.float32)

  h = node_feats
  aggp, degp = _agg_deg_call(h, src, dst, znd, zn, ones)
  h = _tc_layer(h, aggp, degp, Wself0, Wneigh0, bs0, gamma0, beta0)
  aggp = _agg_call(h, src, dst, znd)
  h = _tc_layer(h, aggp, degp, Wself1, Wneigh1, bs1, gamma1, beta1)
  aggp = _agg_call(h, src, dst, znd)
  a_tab, b_tab = _tc_layer_ab(h, aggp, degp, Wself2, Wneigh2, bs2, gamma2,
                              beta2, Wm1[:H], Wm1[H:2 * H], bm1)

  t_flat = _pair_call(a_tab, b_tab, pair_src, pair_dst,
                      edge_feats.reshape(E), Wm1[2 * H], Wm2.reshape(H))
  sel = jnp.kron(jnp.eye(8, dtype=jnp.float32),
                 jnp.ones((16, 1), jnp.float32))
  out8 = _tc_reduce(t_flat.reshape(E * 16 // 128, 128), sel,
                    bm2.reshape(1, 1))
  return out8.reshape(E, 1)


# speedup vs baseline: 1.0241x; 1.0241x over previous
import functools

import jax
import jax.numpy as jnp
from jax import lax
from jax.experimental import pallas as pl
from jax.experimental.pallas import tpu as pltpu
from jax.experimental.pallas import tpu_sc as plsc

N = 10000
E = 320000
D = 128
H = 128

NC = 2
NS = 16
NW = NC * NS
EPW = E // NW
CH = 80
NCHUNK = EPW // CH
CHP = 80
NCHUNKP = EPW // CHP
NGRP = CHP // 16
RPS = 624
RTAIL = N - NS * RPS

_MESH = plsc.VectorSubcoreMesh(
    core_axis_name="c", subcore_axis_name="s", num_cores=NC, num_subcores=NS)


def _agg_pipeline(h_hbm, src_hbm, dst_hbm, agg_sh, base,
                  sidx, didx, rows, gsem, ssem, on_chunk, drain_chunk):
  def start_g(k, b):
    off = pl.multiple_of(base + k * CH, 8)
    pltpu.sync_copy(src_hbm.at[pl.ds(off, CH)], sidx[b])
    pltpu.sync_copy(dst_hbm.at[pl.ds(off, CH)], didx[b])
    pltpu.async_copy(h_hbm.at[sidx[b]], rows[b], gsem[b])

  def scat(k, b):
    pltpu.make_async_copy(h_hbm.at[sidx[b]], rows[b], gsem[b]).wait()
    pltpu.async_copy(rows[b], agg_sh.at[didx[b]], ssem[b], add=True)
    on_chunk(b)

  def drain(b):
    pltpu.make_async_copy(rows[b], agg_sh.at[didx[b]], ssem[b]).wait()
    drain_chunk(b)

  start_g(0, 0)
  start_g(1, 1)
  nstep = (NCHUNK - 1) // 4

  def step(j, carry):
    for i in range(4):
      k = 4 * j + i
      b2 = (i + 2) % 4
      if i in (0, 1):
        @pl.when(j > 0)
        def _():
          drain(b2)
        start_g(k + 2, b2)
      elif i == 2:
        drain(b2)
        start_g(k + 2, b2)
      else:
        @pl.when(j < nstep - 1)
        def _():
          drain(b2)
          start_g(k + 2, b2)
      scat(k, i)
    return carry

  lax.fori_loop(0, nstep, step, 0)
  scat(NCHUNK - 1, (NCHUNK - 1) % 4)
  for b in (1, 2, 3, 0):
    drain(b)


def _agg_deg_body(h_hbm, src_hbm, dst_hbm, znd_hbm, zn_hbm, ones_hbm,
                  agg_out, deg_out,
                  sidx0, sidx1, sidx2, sidx3, didx0, didx1, didx2, didx3,
                  ones_v, rows0, rows1, rows2, rows3,
                  agg_sh, deg_sh,
                  gsem0, gsem1, gsem2, gsem3,
                  ssem0, ssem1, ssem2, ssem3,
                  dsem0, dsem1, dsem2, dsem3):
  cid = lax.axis_index("c")
  sid = lax.axis_index("s")
  wid = cid * NS + sid
  rs = pl.multiple_of(sid * RPS, 8)
  pltpu.sync_copy(znd_hbm.at[pl.ds(rs, RPS)], agg_sh.at[pl.ds(rs, RPS)])
  pltpu.sync_copy(ones_hbm, ones_v)

  @pl.when(sid == 0)
  def _():
    pltpu.sync_copy(zn_hbm, deg_sh)
    tl = pl.multiple_of(NS * RPS, 8)
    pltpu.sync_copy(znd_hbm.at[pl.ds(tl, RTAIL)], agg_sh.at[pl.ds(tl, RTAIL)])

  plsc.subcore_barrier()

  didx = (didx0, didx1, didx2, didx3)
  dsem = (dsem0, dsem1, dsem2, dsem3)

  def on_chunk(b):
    pltpu.async_copy(ones_v, deg_sh.at[didx[b]], dsem[b], add=True)

  def drain_chunk(b):
    pltpu.make_async_copy(ones_v, deg_sh.at[didx[b]], dsem[b]).wait()

  _agg_pipeline(h_hbm, src_hbm, dst_hbm, agg_sh, wid * EPW,
                (sidx0, sidx1, sidx2, sidx3), didx,
                (rows0, rows1, rows2, rows3),
                (gsem0, gsem1, gsem2, gsem3),
                (ssem0, ssem1, ssem2, ssem3),
                on_chunk, drain_chunk)
  plsc.subcore_barrier()
  pltpu.sync_copy(agg_sh.at[pl.ds(rs, RPS)], agg_out.at[cid, pl.ds(rs, RPS)])

  @pl.when(sid == 0)
  def _():
    pltpu.sync_copy(deg_sh, deg_out.at[cid])
    tl = pl.multiple_of(NS * RPS, 8)
    pltpu.sync_copy(agg_sh.at[pl.ds(tl, RTAIL)],
                    agg_out.at[cid, pl.ds(tl, RTAIL)])


def _agg_body(h_hbm, src_hbm, dst_hbm, znd_hbm,
              agg_out,
              sidx0, sidx1, sidx2, sidx3, didx0, didx1, didx2, didx3,
              rows0, rows1, rows2, rows3,
              agg_sh,
              gsem0, gsem1, gsem2, gsem3,
              ssem0, ssem1, ssem2, ssem3):
  cid = lax.axis_index("c")
  sid = lax.axis_index("s")
  wid = cid * NS + sid
  rs = pl.multiple_of(sid * RPS, 8)
  pltpu.sync_copy(znd_hbm.at[pl.ds(rs, RPS)], agg_sh.at[pl.ds(rs, RPS)])

  @pl.when(sid == 0)
  def _():
    tl = pl.multiple_of(NS * RPS, 8)
    pltpu.sync_copy(znd_hbm.at[pl.ds(tl, RTAIL)], agg_sh.at[pl.ds(tl, RTAIL)])

  plsc.subcore_barrier()
  _agg_pipeline(h_hbm, src_hbm, dst_hbm, agg_sh, wid * EPW,
                (sidx0, sidx1, sidx2, sidx3), (didx0, didx1, didx2, didx3),
                (rows0, rows1, rows2, rows3),
                (gsem0, gsem1, gsem2, gsem3),
                (ssem0, ssem1, ssem2, ssem3),
                lambda b: None, lambda b: None)
  plsc.subcore_barrier()
  pltpu.sync_copy(agg_sh.at[pl.ds(rs, RPS)], agg_out.at[cid, pl.ds(rs, RPS)])

  @pl.when(sid == 0)
  def _():
    tl = pl.multiple_of(NS * RPS, 8)
    pltpu.sync_copy(agg_sh.at[pl.ds(tl, RTAIL)],
                    agg_out.at[cid, pl.ds(tl, RTAIL)])


_agg_deg_call = pl.kernel(
    _agg_deg_body,
    out_type=[
        jax.ShapeDtypeStruct((NC, N, D), jnp.float32),
        jax.ShapeDtypeStruct((NC, N), jnp.float32),
    ],
    mesh=_MESH,
    scratch_types=(
        [pltpu.VMEM((CH,), jnp.int32) for _ in range(8)]
        + [pltpu.VMEM((CH,), jnp.float32)]
        + [pltpu.VMEM((CH, D), jnp.float32) for _ in range(4)]
        + [
            pltpu.VMEM_SHARED((N, D), jnp.float32),
            pltpu.VMEM_SHARED((N,), jnp.float32),
        ]
        + [pltpu.SemaphoreType.DMA for _ in range(12)]
    ),
)

_agg_call = pl.kernel(
    _agg_body,
    out_type=jax.ShapeDtypeStruct((NC, N, D), jnp.float32),
    mesh=_MESH,
    scratch_types=(
        [pltpu.VMEM((CH,), jnp.int32) for _ in range(8)]
        + [pltpu.VMEM((CH, D), jnp.float32) for _ in range(4)]
        + [pltpu.VMEM_SHARED((N, D), jnp.float32)]
        + [pltpu.SemaphoreType.DMA for _ in range(8)]
    ),
)


def _pair_body(a_hbm, b_hbm, ps_hbm, pd_hbm, ef_hbm, wrow_hbm, wm2_hbm,
               out_hbm,
               pidx0, pidx1, pidx2, pidx3,
               qidx0, qidx1, qidx2, qidx3,
               rows0, rows1, rows2, rows3,
               efv0, efv1, wv, mv, tv0, tv1,
               gsem0, gsem1, gsem2, gsem3,
               bsem0, bsem1, bsem2, bsem3,
               esem0, esem1, wsem0, wsem1):
  cid = lax.axis_index("c")
  sid = lax.axis_index("s")
  base = (cid * NS + sid) * EPW
  pidx = (pidx0, pidx1, pidx2, pidx3)
  qidx = (qidx0, qidx1, qidx2, qidx3)
  rows = (rows0, rows1, rows2, rows3)
  gsem = (gsem0, gsem1, gsem2, gsem3)
  bsem = (bsem0, bsem1, bsem2, bsem3)
  efv = (efv0, efv1)
  tv = (tv0, tv1)
  esem = (esem0, esem1)
  wsem = (wsem0, wsem1)

  pltpu.sync_copy(wrow_hbm, wv)
  pltpu.sync_copy(wm2_hbm, mv)
  wrow_vals = [wv[pl.ds(16 * i, 16)] for i in range(8)]
  wm2_vals = [mv[pl.ds(16 * i, 16)] for i in range(8)]

  def start_a(k, b):
    off = pl.multiple_of(base + k * CHP, 8)
    pltpu.sync_copy(ps_hbm.at[pl.ds(off, CHP)], pidx[b])
    pltpu.async_copy(a_hbm.at[pidx[b]], rows[b], gsem[b])

  def start_b(k, b, tb):
    off = pl.multiple_of(base + k * CHP, 8)
    pltpu.sync_copy(pd_hbm.at[pl.ds(off, CHP)], qidx[b])
    pltpu.make_async_copy(a_hbm.at[pidx[b]], rows[b], gsem[b]).wait()
    pltpu.async_copy(b_hbm.at[qidx[b]], rows[b], bsem[b], add=True)
    pltpu.async_copy(ef_hbm.at[pl.ds(off, CHP)], efv[tb].at[pl.ds(0, CHP)],
                     esem[tb])

  def compute(k, b, tb, wb_drain):
    off = pl.multiple_of(base + k * CHP, 8)
    pltpu.make_async_copy(b_hbm.at[qidx[b]], rows[b], bsem[b]).wait()
    pltpu.make_async_copy(ef_hbm.at[pl.ds(off, CHP)],
                          efv[tb].at[pl.ds(0, CHP)], esem[tb]).wait()

    def wdrain():
      pltpu.make_async_copy(tv[tb], out_hbm.at[pl.ds(off * 16, CHP * 16)],
                            wsem[tb]).wait()

    if wb_drain is True:
      wdrain()
    elif wb_drain is not False:
      pl.when(wb_drain)(wdrain)

    def group(g, carry):
      ef16 = efv[tb][pl.ds(pl.multiple_of(g * 16, 8), 16)]
      for le in range(16):
        e = g * 16 + le
        ef_spl = jnp.full((16,), ef16[le], jnp.float32)
        acc = None
        for j8 in range(8):
          s = rows[b][e, pl.ds(16 * j8, 16)]
          t = jnp.maximum(s + ef_spl * wrow_vals[j8], 0.0)
          p = t * wm2_vals[j8]
          acc = p if acc is None else acc + p
        tv[tb][pl.ds(pl.multiple_of(e * 16, 8), 16)] = acc
      return carry

    lax.fori_loop(0, NGRP, group, 0)
    pltpu.async_copy(tv[tb], out_hbm.at[pl.ds(off * 16, CHP * 16)], wsem[tb])

  start_a(0, 0)
  start_a(1, 1)

  def step(j, carry):
    for i in range(4):
      k = 4 * j + i
      if i == 3:
        @pl.when(j != (NCHUNKP - 1) // 4 - 1)
        def _():
          start_a(k + 2, (i + 2) % 4)
      else:
        start_a(k + 2, (i + 2) % 4)
      start_b(k, i, i % 2)
      if i == 0:
        @pl.when(j > 0)
        def _():
          compute(k - 1, 3, 1, True)
      elif i == 3:
        compute(k - 1, 2, 0, True)
      else:
        compute(k - 1, i - 1, (i - 1) % 2, j > 0)
    return carry

  lax.fori_loop(0, (NCHUNKP - 1) // 4, step, 0)
  k = NCHUNKP - 1
  off_last = pl.multiple_of(base + k * CHP, 8)
  start_b(k, k % 4, k % 2)
  compute(k - 1, (k - 1) % 4, (k - 1) % 2, True)
  compute(k, k % 4, k % 2, True)
  for tb in (0, 1):
    pltpu.make_async_copy(tv[tb], out_hbm.at[pl.ds(off_last * 16, CHP * 16)],
                          wsem[tb]).wait()


_pair_call = pl.kernel(
    _pair_body,
    out_type=jax.ShapeDtypeStruct((E * 16,), jnp.float32),
    mesh=_MESH,
    scratch_types=(
        [pltpu.VMEM((CHP,), jnp.int32) for _ in range(8)]
        + [pltpu.VMEM((CHP, D), jnp.float32) for _ in range(4)]
        + [
            pltpu.VMEM((128,), jnp.float32),
            pltpu.VMEM((128,), jnp.float32),
            pltpu.VMEM((D,), jnp.float32),
            pltpu.VMEM((D,), jnp.float32),
            pltpu.VMEM((CHP * 16,), jnp.float32),
            pltpu.VMEM((CHP * 16,), jnp.float32),
        ]
        + [pltpu.SemaphoreType.DMA for _ in range(12)]
    ),
)

BR = 2000


def _tred_body(t_ref, s_ref, bm2_ref, out_ref):
  out_ref[...] = (
      jnp.dot(t_ref[...], s_ref[...], preferred_element_type=jnp.float32)
      + bm2_ref[...])


def _tc_reduce(t2, sel, bm2):
  nrow = E * 16 // 128
  return pl.pallas_call(
      _tred_body,
      grid=(nrow // BR,),
      in_specs=[
          pl.BlockSpec((BR, 128), lambda k: (k, 0)),
          pl.BlockSpec((128, 8), lambda k: (0, 0)),
          pl.BlockSpec((1, 1), lambda k: (0, 0)),
      ],
      out_specs=pl.BlockSpec((BR, 8), lambda k: (k, 0)),
      out_shape=jax.ShapeDtypeStruct((nrow, 8), jnp.float32),
  )(t2, sel, bm2)


def _layer_body(h_ref, aggp_ref, degp_ref, ws_ref, wn_ref, b_ref, g_ref,
                be_ref, out_ref):
  agg = aggp_ref[0] + aggp_ref[1]
  deg = degp_ref[0] + degp_ref[1]
  mean = agg / jnp.maximum(deg, 1.0)[:, None]
  z = (jnp.dot(h_ref[...], ws_ref[...], preferred_element_type=jnp.float32)
       + jnp.dot(mean, wn_ref[...], preferred_element_type=jnp.float32)
       + b_ref[...])
  mu = jnp.mean(z, axis=0)
  var = jnp.mean(jnp.square(z - mu), axis=0)
  zn = (z - mu) / jnp.sqrt(var + 1e-5) * g_ref[...] + be_ref[...]
  out_ref[...] = jnp.maximum(zn, 0.0)


def _tc_layer(h, aggp, degp, ws, wn, b, g, be):
  return pl.pallas_call(
      _layer_body,
      out_shape=jax.ShapeDtypeStruct((N, H), jnp.float32),
  )(h, aggp, degp, ws, wn, b, g, be)


def _layer_ab_body(h_ref, aggp_ref, degp_ref, ws_ref, wn_ref, b_ref, g_ref,
                   be_ref, wma_ref, wmb_ref, bm1_ref, a_ref, b_out_ref):
  agg = aggp_ref[0] + aggp_ref[1]
  deg = degp_ref[0] + degp_ref[1]
  mean = agg / jnp.maximum(deg, 1.0)[:, None]
  z = (jnp.dot(h_ref[...], ws_ref[...], preferred_element_type=jnp.float32)
       + jnp.dot(mean, wn_ref[...], preferred_element_type=jnp.float32)
       + b_ref[...])
  mu = jnp.mean(z, axis=0)
  var = jnp.mean(jnp.square(z - mu), axis=0)
  zn = (z - mu) / jnp.sqrt(var + 1e-5) * g_ref[...] + be_ref[...]
  h3 = jnp.maximum(zn, 0.0)
  a_ref[...] = jnp.dot(h3, wma_ref[...], preferred_element_type=jnp.float32)
  b_out_ref[...] = (
      jnp.dot(h3, wmb_ref[...], preferred_element_type=jnp.float32)
      + bm1_ref[...])


def _tc_layer_ab(h, aggp, degp, ws, wn, b, g, be, wma, wmb, bm1):
  return pl.pallas_call(
      _layer_ab_body,
      out_shape=[
          jax.ShapeDtypeStruct((N, H), jnp.float32),
          jax.ShapeDtypeStruct((N, H), jnp.float32),
      ],
  )(h, aggp, degp, ws, wn, b, g, be, wma, wmb, bm1)


@jax.jit
def kernel(node_feats, edge_index, edge_feats, pair_src, pair_dst,
           Wself0, Wneigh0, bs0, gamma0, beta0,
           Wself1, Wneigh1, bs1, gamma1, beta1,
           Wself2, Wneigh2, bs2, gamma2, beta2,
           Wm1, bm1, Wm2, bm2):
  src = edge_index[0]
  dst = edge_index[1]
  znd = jnp.zeros((N, D), jnp.float32)
  zn = jnp.zeros((N,), jnp.float32)
  ones = jnp.ones((CH,), jnp.float32)

  h = node_feats
  aggp, degp = _agg_deg_call(h, src, dst, znd, zn, ones)
  h = _tc_layer(h, aggp, degp, Wself0, Wneigh0, bs0, gamma0, beta0)
  aggp = _agg_call(h, src, dst, znd)
  h = _tc_layer(h, aggp, degp, Wself1, Wneigh1, bs1, gamma1, beta1)
  aggp = _agg_call(h, src, dst, znd)
  a_tab, b_tab = _tc_layer_ab(h, aggp, degp, Wself2, Wneigh2, bs2, gamma2,
                              beta2, Wm1[:H], Wm1[H:2 * H], bm1)

  t_flat = _pair_call(a_tab, b_tab, pair_src, pair_dst,
                      edge_feats.reshape(E), Wm1[2 * H], Wm2.reshape(H))
  sel = jnp.kron(jnp.eye(8, dtype=jnp.float32),
                 jnp.ones((16, 1), jnp.float32))
  out8 = _tc_reduce(t_flat.reshape(E * 16 // 128, 128), sel,
                    bm2.reshape(1, 1))
  return out8.reshape(E, 1)
